# Initial kernel scaffold; baseline (speedup 1.0000x reference)
import jax
import jax.numpy as jnp
from jax.experimental import pallas as pl


def kernel(logits):
    raise NotImplementedError("write your pallas kernel here")


# speedup vs baseline: 29.3846x; 29.3846x over previous
import jax
import jax.numpy as jnp
from jax.experimental import pallas as pl

_R = 0.86
_LANES = 128


def _body(x_ref, o_ref):
    x = x_ref[...]
    m = jnp.max(x, axis=0, keepdims=True)
    shifted = x - m
    e = jnp.exp(shifted)
    s = jnp.sum(e, axis=0, keepdims=True)
    target = _R * s

    lo = jnp.zeros(s.shape, jnp.int32)
    hi = jnp.full(s.shape, 0x3F800000, jnp.int32)

    def step(_, carry):
        lo, hi = carry
        mid = lo + ((hi - lo) >> 1)
        mid_f = jax.lax.bitcast_convert_type(mid, jnp.float32)
        g = jnp.sum(jnp.where(e > mid_f, e, 0.0), axis=0, keepdims=True)
        cond = g < target
        return jnp.where(cond, lo, mid + 1), jnp.where(cond, mid, hi)

    lo, hi = jax.lax.fori_loop(0, 30, step, (lo, hi))
    thr = jax.lax.bitcast_convert_type(lo, jnp.float32)

    o_ref[...] = jnp.where(e >= thr, shifted - jnp.log(s), -70.0)


def kernel(logits):
    B, V, C = logits.shape
    xt = logits.transpose(1, 0, 2).reshape(V, B * C)
    grid = (B * C // _LANES,)
    out = pl.pallas_call(
        _body,
        grid=grid,
        in_specs=[pl.BlockSpec((V, _LANES), lambda c: (0, c))],
        out_specs=pl.BlockSpec((V, _LANES), lambda c: (0, c)),
        out_shape=jax.ShapeDtypeStruct((V, B * C), jnp.float32),
    )(xt)
    return out.reshape(V, B, C).transpose(1, 0, 2)
